# baseline (speedup 1.0000x reference)
import functools

import jax
import jax.numpy as jnp
from jax import lax
from jax.experimental import pallas as pl
from jax.experimental.pallas import tpu as pltpu
from jax.experimental.pallas import tpu_sc as plsc

N = 10000
E = 160000
D = 256
HALF = 128
NC = 2
NS = 16
EPT = E // NS
CH = 128
NBUF = 2
NSEG = 2
SEGC = 40
NCH = NSEG * SEGC
EPAD = NCH * CH
ZPAD = 8
ACC_ROWS = N
RPT = 624
REM = N - NS * RPT


def _sc_agg_body(xh, esrc, edst, out, idx_buf, dst_buf,
                 rows0, acc, sem_g, sem_s):
    c = lax.axis_index("c")
    s = lax.axis_index("s")
    w = c * NS + s
    base = s * RPT
    pltpu.sync_copy(xh.at[pl.ds(c * N + base, RPT)], acc.at[pl.ds(base, RPT)])

    @pl.when(s == 0)
    def _init_tail():
        pltpu.sync_copy(xh.at[pl.ds(c * N + NS * RPT, REM)],
                        acc.at[pl.ds(NS * RPT, REM)])
    plsc.subcore_barrier()

    pltpu.sync_copy(esrc.at[w], idx_buf)
    pltpu.sync_copy(edst.at[w], dst_buf)

    def step_body(j, carry):
        pltpu.async_copy(xh.at[idx_buf.at[j]], rows0, sem_g).wait()
        pltpu.sync_copy(rows0, acc.at[dst_buf.at[j]], add=True)
        return carry

    lax.fori_loop(0, NCH, step_body, 0)
    plsc.subcore_barrier()
    pltpu.sync_copy(acc.at[pl.ds(base, RPT)], out.at[pl.ds(c * N + base, RPT)])

    @pl.when(s == 0)
    def _write_tail():
        pltpu.sync_copy(acc.at[pl.ds(NS * RPT, REM)],
                        out.at[pl.ds(c * N + NS * RPT, REM)])


_sc_aggregate = functools.partial(
    pl.kernel,
    out_type=jax.ShapeDtypeStruct((NC * N, HALF), jnp.float32),
    mesh=plsc.VectorSubcoreMesh(
        core_axis_name="c", subcore_axis_name="s", num_cores=NC, num_subcores=NS
    ),
    scratch_types=[
        pltpu.VMEM((NCH, CH), jnp.int32),
        pltpu.VMEM((NCH, CH), jnp.int32),
        pltpu.VMEM((CH, HALF), jnp.float32),
        pltpu.VMEM_SHARED((ACC_ROWS, HALF), jnp.float32),
        pltpu.SemaphoreType.DMA,
        pltpu.SemaphoreType.DMA,
    ],
)(_sc_agg_body)


BN = 1000


def _mlp_body_split(a_ref, w1_ref, b1_ref, w2_ref, b2_ref, o_ref):
    h = jnp.dot(a_ref[0], w1_ref[0], preferred_element_type=jnp.float32)
    h = h + jnp.dot(a_ref[1], w1_ref[1], preferred_element_type=jnp.float32)
    h = jnp.maximum(h + b1_ref[...], 0.0)
    o_ref[0] = jnp.dot(h, w2_ref[:, :HALF], preferred_element_type=jnp.float32) + b2_ref[:, :HALF]
    o_ref[1] = jnp.dot(h, w2_ref[:, HALF:], preferred_element_type=jnp.float32) + b2_ref[:, HALF:]


def _mlp_body_final(a_ref, w1_ref, b1_ref, w2_ref, b2_ref, o_ref):
    h = jnp.dot(a_ref[0], w1_ref[0], preferred_element_type=jnp.float32)
    h = h + jnp.dot(a_ref[1], w1_ref[1], preferred_element_type=jnp.float32)
    h = jnp.maximum(h + b1_ref[...], 0.0)
    o_ref[...] = jnp.dot(h, w2_ref[...], preferred_element_type=jnp.float32) + b2_ref[...]


def _mlp_call(split, aggh, W1, b1, W2, b2):
    a3 = aggh.reshape(NC, N, HALF)
    w1 = W1.reshape(NC, HALF, D)
    b1r = b1.reshape(1, D)
    b2r = b2.reshape(1, D)
    in_specs = [
        pl.BlockSpec((NC, BN, HALF), lambda i: (0, i, 0)),
        pl.BlockSpec((NC, HALF, D), lambda i: (0, 0, 0)),
        pl.BlockSpec((1, D), lambda i: (0, 0)),
        pl.BlockSpec((D, D), lambda i: (0, 0)),
        pl.BlockSpec((1, D), lambda i: (0, 0)),
    ]
    if split:
        out_shape = jax.ShapeDtypeStruct((NC, N, HALF), jnp.float32)
        out_spec = pl.BlockSpec((NC, BN, HALF), lambda i: (0, i, 0))
        body = _mlp_body_split
    else:
        out_shape = jax.ShapeDtypeStruct((N, D), jnp.float32)
        out_spec = pl.BlockSpec((BN, D), lambda i: (i, 0))
        body = _mlp_body_final
    return pl.pallas_call(
        body,
        grid=(N // BN,),
        in_specs=in_specs,
        out_specs=out_spec,
        out_shape=out_shape,
    )(a3, w1, b1r, W2, b2r)


def kernel(x, edge_index, W1_0, b1_0, W2_0, b2_0, W1_1, b1_1, W2_1, b2_1,
           W1_2, b1_2, W2_2, b2_2):
    src = edge_index[0]
    dst = edge_index[1]
    src2 = jnp.concatenate([src, src + N]).reshape(NC, NS, EPT)
    src2 = jnp.pad(src2, ((0, 0), (0, 0), (0, EPAD - EPT)),
                   constant_values=NC * N)
    esrc = src2.reshape(NC * NS, NCH, CH)
    pad_dst = (jnp.arange(NS, dtype=jnp.int32)[:, None] * RPT
               + jnp.arange(EPAD - EPT, dtype=jnp.int32)[None, :])
    d3 = jnp.concatenate(
        [dst.reshape(NS, EPT),
         pad_dst.astype(jnp.int32)], axis=1)
    d3 = jnp.broadcast_to(d3.reshape(1, NS, EPAD), (NC, NS, EPAD))
    edst = d3.reshape(NC * NS, NCH, CH)

    xh = x.reshape(N, NC, HALF).transpose(1, 0, 2).reshape(NC * N, HALF)
    xh = jnp.concatenate([xh, jnp.zeros((ZPAD, HALF), jnp.float32)])

    params = [(W1_0, b1_0, W2_0, b2_0), (W1_1, b1_1, W2_1, b2_1),
              (W1_2, b1_2, W2_2, b2_2)]
    for l, (W1, b1, W2, b2) in enumerate(params):
        aggh = _sc_aggregate(xh, esrc, edst)
        if l < 2:
            xh = jnp.concatenate(
                [_mlp_call(True, aggh, W1, b1, W2, b2).reshape(NC * N, HALF),
                 jnp.zeros((ZPAD, HALF), jnp.float32)])
        else:
            return _mlp_call(False, aggh, W1, b1, W2, b2)


# speedup vs baseline: 1.4606x; 1.4606x over previous
import functools

import jax
import jax.numpy as jnp
from jax import lax
from jax.experimental import pallas as pl
from jax.experimental.pallas import tpu as pltpu
from jax.experimental.pallas import tpu_sc as plsc

N = 10000
E = 160000
D = 256
HALF = 128
NC = 2
NS = 16
EPT = E // NS
CH = 128
NBUF = 2
NCH = -(-EPT // CH)
EPAD = NCH * CH
ZPAD = 8
ACC_ROWS = N
RPT = 624
REM = N - NS * RPT


def _sc_agg_body(xh, esrc, edst, out, idx_buf, dst_buf,
                 rows0, acc, sem_g, sem_s):
    c = lax.axis_index("c")
    s = lax.axis_index("s")
    w = c * NS + s
    base = s * RPT
    pltpu.sync_copy(xh.at[pl.ds(c * N + base, RPT)], acc.at[pl.ds(base, RPT)])

    @pl.when(s == 0)
    def _init_tail():
        pltpu.sync_copy(xh.at[pl.ds(c * N + NS * RPT, REM)],
                        acc.at[pl.ds(NS * RPT, REM)])
    pltpu.sync_copy(esrc.at[w], idx_buf)
    pltpu.sync_copy(edst.at[w], dst_buf)
    plsc.subcore_barrier()

    def step_body(j, carry):
        pltpu.async_copy(xh.at[idx_buf.at[j]], rows0, sem_g).wait()
        pltpu.sync_copy(rows0, acc.at[dst_buf.at[j]], add=True)
        return carry

    lax.fori_loop(0, NCH, step_body, 0)
    plsc.subcore_barrier()
    pltpu.sync_copy(acc.at[pl.ds(base, RPT)], out.at[pl.ds(c * N + base, RPT)])

    @pl.when(s == 0)
    def _write_tail():
        pltpu.sync_copy(acc.at[pl.ds(NS * RPT, REM)],
                        out.at[pl.ds(c * N + NS * RPT, REM)])


_sc_aggregate = functools.partial(
    pl.kernel,
    out_type=jax.ShapeDtypeStruct((NC * N, HALF), jnp.float32),
    mesh=plsc.VectorSubcoreMesh(
        core_axis_name="c", subcore_axis_name="s", num_cores=NC, num_subcores=NS
    ),
    scratch_types=[
        pltpu.VMEM((NCH, CH), jnp.int32),
        pltpu.VMEM((NCH, CH), jnp.int32),
        pltpu.VMEM((CH, HALF), jnp.float32),
        pltpu.VMEM_SHARED((ACC_ROWS, HALF), jnp.float32),
        pltpu.SemaphoreType.DMA,
        pltpu.SemaphoreType.DMA,
    ],
)(_sc_agg_body)


BN = 1000


def _mlp_body_split(a_ref, w1_ref, b1_ref, w2_ref, b2_ref, o_ref):
    h = jnp.dot(a_ref[0], w1_ref[0], preferred_element_type=jnp.float32)
    h = h + jnp.dot(a_ref[1], w1_ref[1], preferred_element_type=jnp.float32)
    h = jnp.maximum(h + b1_ref[...], 0.0)
    o_ref[0] = jnp.dot(h, w2_ref[:, :HALF], preferred_element_type=jnp.float32) + b2_ref[:, :HALF]
    o_ref[1] = jnp.dot(h, w2_ref[:, HALF:], preferred_element_type=jnp.float32) + b2_ref[:, HALF:]


def _mlp_body_final(a_ref, w1_ref, b1_ref, w2_ref, b2_ref, o_ref):
    h = jnp.dot(a_ref[0], w1_ref[0], preferred_element_type=jnp.float32)
    h = h + jnp.dot(a_ref[1], w1_ref[1], preferred_element_type=jnp.float32)
    h = jnp.maximum(h + b1_ref[...], 0.0)
    o_ref[...] = jnp.dot(h, w2_ref[...], preferred_element_type=jnp.float32) + b2_ref[...]


def _mlp_call(split, aggh, W1, b1, W2, b2):
    a3 = aggh.reshape(NC, N, HALF)
    w1 = W1.reshape(NC, HALF, D)
    b1r = b1.reshape(1, D)
    b2r = b2.reshape(1, D)
    in_specs = [
        pl.BlockSpec((NC, BN, HALF), lambda i: (0, i, 0)),
        pl.BlockSpec((NC, HALF, D), lambda i: (0, 0, 0)),
        pl.BlockSpec((1, D), lambda i: (0, 0)),
        pl.BlockSpec((D, D), lambda i: (0, 0)),
        pl.BlockSpec((1, D), lambda i: (0, 0)),
    ]
    if split:
        out_shape = jax.ShapeDtypeStruct((NC, N, HALF), jnp.float32)
        out_spec = pl.BlockSpec((NC, BN, HALF), lambda i: (0, i, 0))
        body = _mlp_body_split
    else:
        out_shape = jax.ShapeDtypeStruct((N, D), jnp.float32)
        out_spec = pl.BlockSpec((BN, D), lambda i: (i, 0))
        body = _mlp_body_final
    return pl.pallas_call(
        body,
        grid=(N // BN,),
        in_specs=in_specs,
        out_specs=out_spec,
        out_shape=out_shape,
    )(a3, w1, b1r, W2, b2r)


def kernel(x, edge_index, W1_0, b1_0, W2_0, b2_0, W1_1, b1_1, W2_1, b2_1,
           W1_2, b1_2, W2_2, b2_2):
    src = edge_index[0]
    dst = edge_index[1]
    src2 = jnp.concatenate([src, src + N]).reshape(NC, NS, EPT)
    src2 = jnp.pad(src2, ((0, 0), (0, 0), (0, EPAD - EPT)),
                   constant_values=NC * N)
    esrc = src2.reshape(NC * NS, NCH, CH)
    pad_dst = (jnp.arange(NS, dtype=jnp.int32)[:, None] * RPT
               + jnp.arange(EPAD - EPT, dtype=jnp.int32)[None, :])
    d3 = jnp.concatenate(
        [dst.reshape(NS, EPT),
         pad_dst.astype(jnp.int32)], axis=1)
    d3 = jnp.broadcast_to(d3.reshape(1, NS, EPAD), (NC, NS, EPAD))
    edst = d3.reshape(NC * NS, NCH, CH)

    xh = x.reshape(N, NC, HALF).transpose(1, 0, 2).reshape(NC * N, HALF)
    xh = jnp.concatenate([xh, jnp.zeros((ZPAD, HALF), jnp.float32)])

    params = [(W1_0, b1_0, W2_0, b2_0), (W1_1, b1_1, W2_1, b2_1),
              (W1_2, b1_2, W2_2, b2_2)]
    for l, (W1, b1, W2, b2) in enumerate(params):
        aggh = _sc_aggregate(xh, esrc, edst)
        if l < 2:
            xh = jnp.concatenate(
                [_mlp_call(True, aggh, W1, b1, W2, b2).reshape(NC * N, HALF),
                 jnp.zeros((ZPAD, HALF), jnp.float32)])
        else:
            return _mlp_call(False, aggh, W1, b1, W2, b2)


# speedup vs baseline: 2.4285x; 1.6627x over previous
import functools

import jax
import jax.numpy as jnp
from jax import lax
from jax.experimental import pallas as pl
from jax.experimental.pallas import tpu as pltpu
from jax.experimental.pallas import tpu_sc as plsc

N = 10000
E = 160000
D = 256
HALF = 128
NC = 2
NS = 16
EPT = E // NS
CH = 128
NBUF = 2
NCH = -(-EPT // CH)
EPAD = NCH * CH
TR = 512
ACC_ROWS = N + TR
RPT = 624
REM = N - NS * RPT


def _sc_agg_body(xh, esrc, edst, out, idx_buf, dst_buf,
                 rows0, acc, sem_g, sem_s):
    c = lax.axis_index("c")
    s = lax.axis_index("s")
    w = c * NS + s
    base = s * RPT
    pltpu.sync_copy(xh.at[pl.ds(c * N + base, RPT)], acc.at[pl.ds(base, RPT)])

    @pl.when(s == 0)
    def _init_tail():
        pltpu.sync_copy(xh.at[pl.ds(c * N + NS * RPT, REM)],
                        acc.at[pl.ds(NS * RPT, REM)])
    pltpu.sync_copy(esrc.at[w], idx_buf)
    pltpu.sync_copy(edst.at[w], dst_buf)
    plsc.subcore_barrier()

    def step_body(j, carry):
        pltpu.async_copy(xh.at[idx_buf.at[j]], rows0, sem_g).wait()
        pltpu.sync_copy(rows0, acc.at[dst_buf.at[j]], add=True)
        return carry

    lax.fori_loop(0, NCH, step_body, 0)
    plsc.subcore_barrier()
    pltpu.sync_copy(acc.at[pl.ds(base, RPT)], out.at[pl.ds(c * N + base, RPT)])

    @pl.when(s == 0)
    def _write_tail():
        pltpu.sync_copy(acc.at[pl.ds(NS * RPT, REM)],
                        out.at[pl.ds(c * N + NS * RPT, REM)])


_sc_aggregate = functools.partial(
    pl.kernel,
    out_type=jax.ShapeDtypeStruct((NC * N, HALF), jnp.float32),
    mesh=plsc.VectorSubcoreMesh(
        core_axis_name="c", subcore_axis_name="s", num_cores=NC, num_subcores=NS
    ),
    scratch_types=[
        pltpu.VMEM((NCH, CH), jnp.int32),
        pltpu.VMEM((NCH, CH), jnp.int32),
        pltpu.VMEM((CH, HALF), jnp.float32),
        pltpu.VMEM_SHARED((ACC_ROWS, HALF), jnp.float32),
        pltpu.SemaphoreType.DMA,
        pltpu.SemaphoreType.DMA,
    ],
)(_sc_agg_body)


BN = 1000


def _mlp_body_split(a_ref, w1_ref, b1_ref, w2_ref, b2_ref, o_ref):
    h = jnp.dot(a_ref[0], w1_ref[0], preferred_element_type=jnp.float32)
    h = h + jnp.dot(a_ref[1], w1_ref[1], preferred_element_type=jnp.float32)
    h = jnp.maximum(h + b1_ref[...], 0.0)
    o_ref[0] = jnp.dot(h, w2_ref[:, :HALF], preferred_element_type=jnp.float32) + b2_ref[:, :HALF]
    o_ref[1] = jnp.dot(h, w2_ref[:, HALF:], preferred_element_type=jnp.float32) + b2_ref[:, HALF:]


def _mlp_body_final(a_ref, w1_ref, b1_ref, w2_ref, b2_ref, o_ref):
    h = jnp.dot(a_ref[0], w1_ref[0], preferred_element_type=jnp.float32)
    h = h + jnp.dot(a_ref[1], w1_ref[1], preferred_element_type=jnp.float32)
    h = jnp.maximum(h + b1_ref[...], 0.0)
    o_ref[...] = jnp.dot(h, w2_ref[...], preferred_element_type=jnp.float32) + b2_ref[...]


def _mlp_call(split, aggh, W1, b1, W2, b2):
    a3 = aggh.reshape(NC, N, HALF)
    w1 = W1.reshape(NC, HALF, D)
    b1r = b1.reshape(1, D)
    b2r = b2.reshape(1, D)
    in_specs = [
        pl.BlockSpec((NC, BN, HALF), lambda i: (0, i, 0)),
        pl.BlockSpec((NC, HALF, D), lambda i: (0, 0, 0)),
        pl.BlockSpec((1, D), lambda i: (0, 0)),
        pl.BlockSpec((D, D), lambda i: (0, 0)),
        pl.BlockSpec((1, D), lambda i: (0, 0)),
    ]
    if split:
        out_shape = jax.ShapeDtypeStruct((NC, N, HALF), jnp.float32)
        out_spec = pl.BlockSpec((NC, BN, HALF), lambda i: (0, i, 0))
        body = _mlp_body_split
    else:
        out_shape = jax.ShapeDtypeStruct((N, D), jnp.float32)
        out_spec = pl.BlockSpec((BN, D), lambda i: (i, 0))
        body = _mlp_body_final
    return pl.pallas_call(
        body,
        grid=(N // BN,),
        in_specs=in_specs,
        out_specs=out_spec,
        out_shape=out_shape,
    )(a3, w1, b1r, W2, b2r)


def kernel(x, edge_index, W1_0, b1_0, W2_0, b2_0, W1_1, b1_1, W2_1, b2_1,
           W1_2, b1_2, W2_2, b2_2):
    src = edge_index[0]
    dst = edge_index[1]
    PADN = EPAD - EPT
    pad_rng = jnp.arange(PADN, dtype=jnp.int32)
    pad_src = (jnp.arange(NS, dtype=jnp.int32)[None, :, None] * RPT
               + pad_rng[None, None, :]
               + jnp.arange(NC, dtype=jnp.int32)[:, None, None] * N)
    src2 = jnp.concatenate(
        [jnp.concatenate([src, src + N]).reshape(NC, NS, EPT),
         pad_src], axis=2)
    esrc = src2.reshape(NC * NS, NCH, CH)
    w_ids = jnp.arange(NC * NS, dtype=jnp.int32).reshape(NC, NS)
    pad_dst = N + (w_ids[:, :, None] * PADN + pad_rng[None, None, :]) % TR
    d3 = jnp.concatenate(
        [jnp.broadcast_to(dst.reshape(1, NS, EPT), (NC, NS, EPT)),
         pad_dst.astype(jnp.int32)], axis=2)
    edst = d3.reshape(NC * NS, NCH, CH)

    xh = x.reshape(N, NC, HALF).transpose(1, 0, 2).reshape(NC * N, HALF)

    params = [(W1_0, b1_0, W2_0, b2_0), (W1_1, b1_1, W2_1, b2_1),
              (W1_2, b1_2, W2_2, b2_2)]
    for l, (W1, b1, W2, b2) in enumerate(params):
        aggh = _sc_aggregate(xh, esrc, edst)
        if l < 2:
            xh = _mlp_call(True, aggh, W1, b1, W2, b2).reshape(NC * N, HALF)
        else:
            return _mlp_call(False, aggh, W1, b1, W2, b2)


# speedup vs baseline: 3.0910x; 1.2728x over previous
import functools

import jax
import jax.numpy as jnp
from jax import lax
from jax.experimental import pallas as pl
from jax.experimental.pallas import tpu as pltpu
from jax.experimental.pallas import tpu_sc as plsc

N = 10000
E = 160000
D = 256
HALF = 128
NC = 2
NS = 16
EPT = E // NS
CH = 128
NBUF = 2
NSEG = 2
SEGC = 40
NCH = NSEG * SEGC
EPAD = NCH * CH
TR = 512
ACC_ROWS = N + TR
RPT = 624
REM = N - NS * RPT


def _sc_agg_body(xh, esrc, edst, out, idx_buf, dst_buf,
                 rows0, rows1, acc, sem_g0, sem_g1, sem_s0, sem_s1):
    c = lax.axis_index("c")
    s = lax.axis_index("s")
    w = c * NS + s
    base = s * RPT
    pltpu.sync_copy(xh.at[pl.ds(c * N + base, RPT)], acc.at[pl.ds(base, RPT)])

    @pl.when(s == 0)
    def _init_tail():
        pltpu.sync_copy(xh.at[pl.ds(c * N + NS * RPT, REM)],
                        acc.at[pl.ds(NS * RPT, REM)])
    plsc.subcore_barrier()
    rows = (rows0, rows1)
    sem_g = (sem_g0, sem_g1)
    sem_s = (sem_s0, sem_s1)

    def g_start(j, b):
        pltpu.make_async_copy(xh.at[idx_buf.at[j]], rows[b], sem_g[b]).start()

    def g_wait(j, b):
        pltpu.make_async_copy(xh.at[idx_buf.at[j]], rows[b], sem_g[b]).wait()

    def s_start(j, b):
        pltpu.make_async_copy(rows[b], acc.at[dst_buf.at[j]],
                              sem_s[b]).start(add=True)

    def s_wait(j, b):
        pltpu.make_async_copy(rows[b], acc.at[dst_buf.at[j]], sem_s[b]).wait()

    for h in range(NSEG):
        pltpu.sync_copy(esrc.at[w, pl.ds(h * SEGC, SEGC)], idx_buf)
        pltpu.sync_copy(edst.at[w, pl.ds(h * SEGC, SEGC)], dst_buf)
        g_start(0, 0)

        def step_body(t, carry):
            for b in range(NBUF):
                j = t * NBUF + b
                g_wait(j, b)
                s_start(j, b)

                @pl.when(j >= 1)
                def _():
                    s_wait(j - 1, 1 - b)

                @pl.when(j + 1 < SEGC)
                def _():
                    g_start(j + 1, 1 - b)
            return carry

        lax.fori_loop(0, SEGC // NBUF, step_body, 0)
        s_wait(SEGC - 1, (SEGC - 1) % NBUF)
    plsc.subcore_barrier()
    pltpu.sync_copy(acc.at[pl.ds(base, RPT)], out.at[pl.ds(c * N + base, RPT)])

    @pl.when(s == 0)
    def _write_tail():
        pltpu.sync_copy(acc.at[pl.ds(NS * RPT, REM)],
                        out.at[pl.ds(c * N + NS * RPT, REM)])


_sc_aggregate = functools.partial(
    pl.kernel,
    out_type=jax.ShapeDtypeStruct((NC * N, HALF), jnp.float32),
    mesh=plsc.VectorSubcoreMesh(
        core_axis_name="c", subcore_axis_name="s", num_cores=NC, num_subcores=NS
    ),
    scratch_types=[
        pltpu.VMEM((SEGC, CH), jnp.int32),
        pltpu.VMEM((SEGC, CH), jnp.int32),
        pltpu.VMEM((CH, HALF), jnp.float32),
        pltpu.VMEM((CH, HALF), jnp.float32),
        pltpu.VMEM_SHARED((ACC_ROWS, HALF), jnp.float32),
        pltpu.SemaphoreType.DMA,
        pltpu.SemaphoreType.DMA,
        pltpu.SemaphoreType.DMA,
        pltpu.SemaphoreType.DMA,
    ],
)(_sc_agg_body)


BN = 1000


def _mlp_body_split(a_ref, w1_ref, b1_ref, w2_ref, b2_ref, o_ref):
    h = jnp.dot(a_ref[0], w1_ref[0], preferred_element_type=jnp.float32)
    h = h + jnp.dot(a_ref[1], w1_ref[1], preferred_element_type=jnp.float32)
    h = jnp.maximum(h + b1_ref[...], 0.0)
    o_ref[0] = jnp.dot(h, w2_ref[:, :HALF], preferred_element_type=jnp.float32) + b2_ref[:, :HALF]
    o_ref[1] = jnp.dot(h, w2_ref[:, HALF:], preferred_element_type=jnp.float32) + b2_ref[:, HALF:]


def _mlp_body_final(a_ref, w1_ref, b1_ref, w2_ref, b2_ref, o_ref):
    h = jnp.dot(a_ref[0], w1_ref[0], preferred_element_type=jnp.float32)
    h = h + jnp.dot(a_ref[1], w1_ref[1], preferred_element_type=jnp.float32)
    h = jnp.maximum(h + b1_ref[...], 0.0)
    o_ref[...] = jnp.dot(h, w2_ref[...], preferred_element_type=jnp.float32) + b2_ref[...]


def _mlp_call(split, aggh, W1, b1, W2, b2):
    a3 = aggh.reshape(NC, N, HALF)
    w1 = W1.reshape(NC, HALF, D)
    b1r = b1.reshape(1, D)
    b2r = b2.reshape(1, D)
    in_specs = [
        pl.BlockSpec((NC, BN, HALF), lambda i: (0, i, 0)),
        pl.BlockSpec((NC, HALF, D), lambda i: (0, 0, 0)),
        pl.BlockSpec((1, D), lambda i: (0, 0)),
        pl.BlockSpec((D, D), lambda i: (0, 0)),
        pl.BlockSpec((1, D), lambda i: (0, 0)),
    ]
    if split:
        out_shape = jax.ShapeDtypeStruct((NC, N, HALF), jnp.float32)
        out_spec = pl.BlockSpec((NC, BN, HALF), lambda i: (0, i, 0))
        body = _mlp_body_split
    else:
        out_shape = jax.ShapeDtypeStruct((N, D), jnp.float32)
        out_spec = pl.BlockSpec((BN, D), lambda i: (i, 0))
        body = _mlp_body_final
    return pl.pallas_call(
        body,
        grid=(N // BN,),
        in_specs=in_specs,
        out_specs=out_spec,
        out_shape=out_shape,
    )(a3, w1, b1r, W2, b2r)


def kernel(x, edge_index, W1_0, b1_0, W2_0, b2_0, W1_1, b1_1, W2_1, b2_1,
           W1_2, b1_2, W2_2, b2_2):
    src = edge_index[0]
    dst = edge_index[1]
    PADN = EPAD - EPT
    pad_rng = jnp.arange(PADN, dtype=jnp.int32)
    pad_src = (jnp.arange(NS, dtype=jnp.int32)[None, :, None] * RPT
               + pad_rng[None, None, :]
               + jnp.arange(NC, dtype=jnp.int32)[:, None, None] * N)
    src2 = jnp.concatenate(
        [jnp.concatenate([src, src + N]).reshape(NC, NS, EPT),
         pad_src], axis=2)
    esrc = src2.reshape(NC * NS, NCH, CH)
    w_ids = jnp.arange(NC * NS, dtype=jnp.int32).reshape(NC, NS)
    pad_dst = N + (w_ids[:, :, None] * PADN + pad_rng[None, None, :]) % TR
    d3 = jnp.concatenate(
        [jnp.broadcast_to(dst.reshape(1, NS, EPT), (NC, NS, EPT)),
         pad_dst.astype(jnp.int32)], axis=2)
    edst = d3.reshape(NC * NS, NCH, CH)

    xh = x.reshape(N, NC, HALF).transpose(1, 0, 2).reshape(NC * N, HALF)

    params = [(W1_0, b1_0, W2_0, b2_0), (W1_1, b1_1, W2_1, b2_1),
              (W1_2, b1_2, W2_2, b2_2)]
    for l, (W1, b1, W2, b2) in enumerate(params):
        aggh = _sc_aggregate(xh, esrc, edst)
        if l < 2:
            xh = _mlp_call(True, aggh, W1, b1, W2, b2).reshape(NC * N, HALF)
        else:
            return _mlp_call(False, aggh, W1, b1, W2, b2)
